# Initial kernel scaffold; baseline (speedup 1.0000x reference)
import jax
import jax.numpy as jnp
from jax.experimental import pallas as pl


def kernel(x, edge_index, W1, b1, W2, b2):
    raise NotImplementedError("write your pallas kernel here")


# speedup vs baseline: 14.2888x; 14.2888x over previous
import functools

import jax
import jax.numpy as jnp
from jax import lax
from jax.experimental import pallas as pl
from jax.experimental.pallas import tpu as pltpu
from jax.experimental.pallas import tpu_sc as plsc

NC = 2
NS = 16
NT = NC * NS
CHUNK = 128


def _make_deg(NP, C):
    mesh = plsc.VectorSubcoreMesh(core_axis_name="c", subcore_axis_name="s", num_cores=NC, num_subcores=NS)
    RPT = NP // NS

    @functools.partial(
        pl.kernel,
        out_type=jax.ShapeDtypeStruct((NC, NP, 8), jnp.float32),
        mesh=mesh,
        compiler_params=pltpu.CompilerParams(use_tc_tiling_on_sc=False),
        scratch_types=[
            pltpu.VMEM((C, CHUNK), jnp.int32),
            pltpu.VMEM((CHUNK, 8), jnp.float32),
            pltpu.VMEM_SHARED((NP, 8), jnp.float32),
        ],
    )
    def deg(dsts_hbm, ones_hbm, zero_hbm, out_hbm, dst_v, ones_v, acc):
        cid = lax.axis_index("c")
        sid = lax.axis_index("s")
        t = cid * NS + sid
        pltpu.sync_copy(dsts_hbm.at[t], dst_v)
        pltpu.sync_copy(zero_hbm, ones_v)
        base = sid * RPT
        for kk in range(RPT // CHUNK):
            pltpu.sync_copy(ones_v, acc.at[pl.ds(base + kk * CHUNK, CHUNK)])
        pltpu.sync_copy(ones_hbm, ones_v)
        plsc.subcore_barrier()

        def body(j, carry):
            pltpu.sync_copy(ones_v, acc.at[dst_v.at[j]], add=True)
            return carry

        lax.fori_loop(0, C, body, 0)
        plsc.subcore_barrier()
        pltpu.sync_copy(acc.at[pl.ds(base, RPT)],
                        out_hbm.at[cid, pl.ds(base, RPT)])

    return deg


def _make_agg(NP, D, C):
    mesh = plsc.VectorSubcoreMesh(core_axis_name="c", subcore_axis_name="s", num_cores=NC, num_subcores=NS)
    RPT = NP // NS

    @functools.partial(
        pl.kernel,
        out_type=jax.ShapeDtypeStruct((NC, NP, D), jnp.float32),
        mesh=mesh,
        compiler_params=pltpu.CompilerParams(use_tc_tiling_on_sc=False),
        scratch_types=[
            pltpu.VMEM((C, CHUNK), jnp.int32),
            pltpu.VMEM((C, CHUNK), jnp.int32),
            pltpu.VMEM((CHUNK, D), jnp.float32),
            pltpu.VMEM_SHARED((NP, D), jnp.float32),
        ],
    )
    def agg(g_hbm, srcs_hbm, dsts_hbm, zero_hbm, out_hbm,
            src_v, dst_v, rows_v, acc):
        cid = lax.axis_index("c")
        sid = lax.axis_index("s")
        t = cid * NS + sid
        pltpu.sync_copy(srcs_hbm.at[t], src_v)
        pltpu.sync_copy(dsts_hbm.at[t], dst_v)
        pltpu.sync_copy(zero_hbm, rows_v)
        base = sid * RPT
        for kk in range(RPT // CHUNK):
            pltpu.sync_copy(rows_v, acc.at[pl.ds(base + kk * CHUNK, CHUNK)])
        plsc.subcore_barrier()

        def body(j, carry):
            pltpu.sync_copy(g_hbm.at[src_v.at[j]], rows_v)
            pltpu.sync_copy(rows_v, acc.at[dst_v.at[j]], add=True)
            return carry

        lax.fori_loop(0, C, body, 0)
        plsc.subcore_barrier()
        pltpu.sync_copy(acc.at[pl.ds(base, RPT)],
                        out_hbm.at[cid, pl.ds(base, RPT)])

    return agg


def _tc_layer1(degp, xp, W1, NP, BR):
    D_IN, D_H = W1.shape

    def body(deg_ref, x_ref, w_ref, g_ref, dis_ref):
        d = 1.0 + deg_ref[0] + deg_ref[1]
        dis = lax.rsqrt(d)[:, None]
        dis_ref[...] = dis
        h = jnp.dot(x_ref[...], w_ref[...], preferred_element_type=jnp.float32)
        g_ref[...] = h * dis

    return pl.pallas_call(
        body,
        grid=(NP // BR,),
        in_specs=[
            pl.BlockSpec((NC, BR), lambda i: (0, i)),
            pl.BlockSpec((BR, D_IN), lambda i: (i, 0)),
            pl.BlockSpec((D_IN, D_H), lambda i: (0, 0)),
        ],
        out_specs=[
            pl.BlockSpec((BR, D_H), lambda i: (i, 0)),
            pl.BlockSpec((BR, 1), lambda i: (i, 0)),
        ],
        out_shape=[
            jax.ShapeDtypeStruct((NP, D_H), jnp.float32),
            jax.ShapeDtypeStruct((NP, 1), jnp.float32),
        ],
    )(degp, xp, W1)


def _tc_layer2(parts, g1, dis, b1, W2p, NP, BR):
    D_H, D_O = W2p.shape

    def body(p_ref, g1_ref, dis_ref, b1_ref, w_ref, g2_ref):
        dis = dis_ref[...]
        s = p_ref[0] + p_ref[1] + g1_ref[...]
        h2 = jnp.maximum(s * dis + b1_ref[...][None, :], 0.0)
        m = jnp.dot(h2, w_ref[...], preferred_element_type=jnp.float32)
        g2_ref[...] = m * dis

    return pl.pallas_call(
        body,
        grid=(NP // BR,),
        in_specs=[
            pl.BlockSpec((NC, BR, D_H), lambda i: (0, i, 0)),
            pl.BlockSpec((BR, D_H), lambda i: (i, 0)),
            pl.BlockSpec((BR, 1), lambda i: (i, 0)),
            pl.BlockSpec((D_H,), lambda i: (0,)),
            pl.BlockSpec((D_H, D_O), lambda i: (0, 0)),
        ],
        out_specs=pl.BlockSpec((BR, D_O), lambda i: (i, 0)),
        out_shape=jax.ShapeDtypeStruct((NP, D_O), jnp.float32),
    )(parts, g1, dis, b1, W2p)


def _tc_final(parts, g2, dis, b2p, NP, BR):
    D_O = b2p.shape[0]

    def body(p_ref, g2_ref, dis_ref, b2_ref, o_ref):
        s = p_ref[0] + p_ref[1] + g2_ref[...]
        o_ref[...] = s * dis_ref[...] + b2_ref[...][None, :]

    return pl.pallas_call(
        body,
        grid=(NP // BR,),
        in_specs=[
            pl.BlockSpec((NC, BR, D_O), lambda i: (0, i, 0)),
            pl.BlockSpec((BR, D_O), lambda i: (i, 0)),
            pl.BlockSpec((BR, 1), lambda i: (i, 0)),
            pl.BlockSpec((D_O,), lambda i: (0,)),
        ],
        out_specs=pl.BlockSpec((BR, D_O), lambda i: (i, 0)),
        out_shape=jax.ShapeDtypeStruct((NP, D_O), jnp.float32),
    )(parts, g2, dis, b2p)


@jax.jit
def _run(x, edge_index, W1, b1, W2, b2):
    N = x.shape[0]
    E = edge_index.shape[1]
    NP = ((N + NS * CHUNK - 1) // (NS * CHUNK)) * (NS * CHUNK)
    C = (E + NT * CHUNK - 1) // (NT * CHUNK)
    EP = NT * C * CHUNK
    BR = NP // 16

    src = edge_index[0].astype(jnp.int32)
    dst = edge_index[1].astype(jnp.int32)
    src_p = jnp.concatenate(
        [src, jnp.zeros((EP - E,), jnp.int32)]).reshape(NT, C, CHUNK)
    dst_p = jnp.concatenate(
        [dst, jnp.full((EP - E,), NP - 1, jnp.int32)]).reshape(NT, C, CHUNK)
    xp = jnp.pad(x, ((0, NP - N), (0, 0)))

    D_H = W1.shape[1]
    D_O2 = 16
    W2p = jnp.pad(W2, ((0, 0), (0, D_O2 - W2.shape[1])))
    b2p = jnp.pad(b2, (0, D_O2 - b2.shape[0]))

    ones8 = jnp.ones((CHUNK, 8), jnp.float32)
    zero8 = jnp.zeros((CHUNK, 8), jnp.float32)
    zeroH = jnp.zeros((CHUNK, D_H), jnp.float32)
    zeroO = jnp.zeros((CHUNK, D_O2), jnp.float32)

    degp8 = _make_deg(NP, C)(dst_p, ones8, zero8)
    degp = degp8[:, :, 0]

    g1, dis = _tc_layer1(degp, xp, W1, NP, BR)
    parts1 = _make_agg(NP, D_H, C)(g1, src_p, dst_p, zeroH)
    g2 = _tc_layer2(parts1, g1, dis, b1, W2p, NP, BR)
    parts2 = _make_agg(NP, D_O2, C)(g2, src_p, dst_p, zeroO)
    out16 = _tc_final(parts2, g2, dis, b2p, NP, BR)
    return out16[:N, :W2.shape[1]]


def kernel(x, edge_index, W1, b1, W2, b2):
    return _run(x, edge_index, W1, b1, W2, b2)
